# baseline (speedup 1.0000x reference)
import functools
import jax
import jax.numpy as jnp
from jax import lax
from jax.experimental import pallas as pl
from jax.experimental.pallas import tpu as pltpu
from jax.experimental.pallas import tpu_sc as plsc

NC = 2
NS = 16
L = 16


def _sc_scatter_fn(N, E, D, B, x_hbm, src_hbm, dst_hbm, ew_hbm, out_hbm,
                   srcs_v, dsts_v, ws_v, dst0, dst1, rows0, rows1, acc_sh,
                   sem_g0, sem_g1, sem_s0, sem_s1):
    NW = NC * NS
    e_per_tile = E // NW
    nbatch = e_per_tile // B
    npair = nbatch // 2
    nchunk_rows = N // B
    nround = pl.cdiv(nchunk_rows, NS)
    nchunk = D // L

    c = lax.axis_index("c")
    s = lax.axis_index("s")
    wid = s * NC + c

    zeros = jnp.zeros((L,), jnp.float32)

    def zero_body(t, _):
        r = t // nchunk
        j = t % nchunk
        rows0[r, pl.ds(j * L, L)] = zeros
        return _

    lax.fori_loop(0, B * nchunk, zero_body, None)

    def acczero_body(t, _):
        chunk = s + NS * t

        @pl.when(chunk < nchunk_rows)
        def _():
            pltpu.sync_copy(rows0, acc_sh.at[pl.ds(chunk * B, B)])

        return _

    lax.fori_loop(0, nround, acczero_body, None)
    plsc.subcore_barrier()

    ebase = wid * e_per_tile
    pltpu.sync_copy(src_hbm.at[pl.ds(ebase, e_per_tile)], srcs_v)
    pltpu.sync_copy(dst_hbm.at[pl.ds(ebase, e_per_tile)], dsts_v)
    pltpu.sync_copy(ew_hbm.at[pl.ds(ebase, e_per_tile)], ws_v)

    def gather(b, rows, sem):
        pltpu.async_copy(x_hbm.at[srcs_v.at[pl.ds(b * B, B)]], rows, sem)

    def gather_wait(rows, sem):
        pltpu.make_async_copy(x_hbm.at[pl.ds(0, B)], rows, sem).wait()

    def scale(b, rows):
        @plsc.parallel_loop(0, B // L)
        def _(g):
            wgrp = ws_v[pl.ds(b * B + g * L, L)]
            for i in range(L):
                e = g * L + i
                wv = jnp.full((L,), wgrp[i], jnp.float32)
                for j in range(nchunk):
                    sl = pl.ds(j * L, L)
                    rows[e, sl] = rows[e, sl] * wv

    def scatter(b, rows, dst_v, sem):
        for k in range(B // L):
            dst_v[pl.ds(k * L, L)] = dsts_v[pl.ds(b * B + k * L, L)]
        pltpu.async_copy(rows, acc_sh.at[dst_v], sem, add=True)

    def scatter_wait(rows, sem):
        pltpu.make_async_copy(rows, acc_sh.at[pl.ds(0, B)], sem).wait()

    gather(0, rows0, sem_g0)

    def pair_body(t, _):
        b0 = 2 * t
        @pl.when(t > 0)
        def _():
            scatter_wait(rows1, sem_s1)
        gather(b0 + 1, rows1, sem_g1)
        gather_wait(rows0, sem_g0)
        scale(b0, rows0)
        scatter(b0, rows0, dst0, sem_s0)
        gather_wait(rows1, sem_g1)
        scale(b0 + 1, rows1)
        scatter(b0 + 1, rows1, dst1, sem_s1)
        scatter_wait(rows0, sem_s0)
        gather(b0 + 2, rows0, sem_g0)
        return _

    lax.fori_loop(0, npair, pair_body, None)
    blast = nbatch - 1
    scatter_wait(rows1, sem_s1)
    gather_wait(rows0, sem_g0)
    scale(blast, rows0)
    scatter(blast, rows0, dst0, sem_s0)
    scatter_wait(rows0, sem_s0)

    plsc.subcore_barrier()

    def wb_body(t, _):
        chunk = s + NS * t

        @pl.when(chunk < nchunk_rows)
        def _():
            r0 = chunk * B
            pltpu.sync_copy(acc_sh.at[pl.ds(r0, B)], rows0)
            pltpu.sync_copy(rows0, out_hbm.at[c, pl.ds(r0, B)])

        return _

    lax.fori_loop(0, nround, wb_body, None)


@functools.partial(jax.jit, static_argnames=("N", "E", "D"))
def _sc_scatter(x, src, dst, ew, N, E, D):
    B = 80
    NW = NC * NS
    NB = E // (NW * B)
    mesh = plsc.VectorSubcoreMesh(
        core_axis_name="c", subcore_axis_name="s",
        num_cores=NC, num_subcores=NS)
    f = pl.kernel(
        functools.partial(_sc_scatter_fn, N, E, D, B),
        out_type=jax.ShapeDtypeStruct((NC, N, D), jnp.float32),
        mesh=mesh,
        scratch_types=[
            pltpu.VMEM((E // NW,), jnp.int32),
            pltpu.VMEM((E // NW,), jnp.int32),
            pltpu.VMEM((E // NW,), jnp.float32),
            pltpu.VMEM((B,), jnp.int32),
            pltpu.VMEM((B,), jnp.int32),
            pltpu.VMEM((B, D), jnp.float32),
            pltpu.VMEM((B, D), jnp.float32),
            pltpu.VMEM_SHARED((N, D), jnp.float32),
            pltpu.SemaphoreType.DMA,
            pltpu.SemaphoreType.DMA,
            pltpu.SemaphoreType.DMA,
            pltpu.SemaphoreType.DMA,
        ],
    )
    return f(x, src, dst, ew)


def _tc_fn(p_ref, w_ref, b_ref, o_ref):
    a = p_ref[0] + p_ref[1]
    o_ref[...] = jnp.dot(a, w_ref[...],
                         preferred_element_type=jnp.float32) + b_ref[...]


@functools.partial(jax.jit, static_argnames=("bn",))
def _tc_finish(partials, weight, bias2d, bn):
    N, D = partials.shape[1], partials.shape[2]
    DO = weight.shape[1]
    grid = (N // bn,)
    return pl.pallas_call(
        _tc_fn,
        grid=grid,
        in_specs=[
            pl.BlockSpec((NC, bn, D), lambda i: (0, i, 0)),
            pl.BlockSpec((D, DO), lambda i: (0, 0)),
            pl.BlockSpec((1, DO), lambda i: (0, 0)),
        ],
        out_specs=pl.BlockSpec((bn, DO), lambda i: (i, 0)),
        out_shape=jax.ShapeDtypeStruct((N, DO), jnp.float32),
    )(partials, weight, bias2d)


def kernel(x, edge_index, edge_weight, weight, bias):
    N, D = x.shape
    E = edge_index.shape[1]
    ew = edge_weight.reshape(-1)
    src = edge_index[0]
    dst = edge_index[1]
    partials = _sc_scatter(x, src, dst, ew, N=N, E=E, D=D)
    return _tc_finish(partials, weight, bias.reshape(1, -1), bn=2000)


# speedup vs baseline: 1.2417x; 1.2417x over previous
import functools
import jax
import jax.numpy as jnp
from jax import lax
from jax.experimental import pallas as pl
from jax.experimental.pallas import tpu as pltpu
from jax.experimental.pallas import tpu_sc as plsc

NC = 2
NS = 16
L = 16


def _sc_scatter_fn(N, E, D, B, x_hbm, src_hbm, dst_hbm, ew_hbm, out_hbm,
                   srcs_v, dst0, dst1, dst2, wv0, wv1, wv2,
                   rows0, rows1, rows2, acc_sh,
                   sg0, sg1, sg2, sd0, sd1, sd2, sw0, sw1, sw2,
                   ss0, ss1, ss2):
    NW = NC * NS
    e_per_tile = E // NW
    nbatch = e_per_tile // B
    ntriple = (nbatch - 2) // 3
    nchunk_rows = N // B
    nround = pl.cdiv(nchunk_rows, NS)
    nchunk = D // L

    c = lax.axis_index("c")
    s = lax.axis_index("s")
    wid = s * NC + c

    zeros = jnp.zeros((L,), jnp.float32)

    def zero_body(t, _):
        r = t // nchunk
        j = t % nchunk
        rows0[r, pl.ds(j * L, L)] = zeros
        return _

    lax.fori_loop(0, B * nchunk, zero_body, None)

    def acczero_body(t, _):
        chunk = s + NS * t

        @pl.when(chunk < nchunk_rows)
        def _():
            pltpu.sync_copy(rows0, acc_sh.at[pl.ds(chunk * B, B)])

        return _

    lax.fori_loop(0, nround, acczero_body, None)
    plsc.subcore_barrier()

    ebase = wid * e_per_tile
    pltpu.sync_copy(src_hbm.at[pl.ds(ebase, e_per_tile)], srcs_v)

    def gather(b, rows, sem):
        pltpu.async_copy(x_hbm.at[srcs_v.at[pl.ds(b * B, B)]], rows, sem)

    def gather_wait(rows, sem):
        pltpu.make_async_copy(x_hbm.at[pl.ds(0, B)], rows, sem).wait()

    def dstload(b, dst_v, sem):
        pltpu.async_copy(dst_hbm.at[pl.ds(ebase + b * B, B)], dst_v, sem)

    def dst_wait(dst_v, sem):
        pltpu.make_async_copy(dst_hbm.at[pl.ds(0, B)], dst_v, sem).wait()

    def wload(b, w_v, sem):
        pltpu.async_copy(ew_hbm.at[pl.ds(ebase + b * B, B)], w_v, sem)

    def w_wait(w_v, sem):
        pltpu.make_async_copy(ew_hbm.at[pl.ds(0, B)], w_v, sem).wait()

    def scale(rows, w_v):
        def scale_body(g, _):
            wgrp = w_v[pl.ds(g * L, L)]
            for i in range(L):
                e = g * L + i
                wv = jnp.full((L,), wgrp[i], jnp.float32)
                for j in range(nchunk):
                    sl = pl.ds(j * L, L)
                    rows[e, sl] = rows[e, sl] * wv
            return _

        lax.fori_loop(0, B // L, scale_body, None)

    def scatter(rows, dst_v, sem):
        pltpu.async_copy(rows, acc_sh.at[dst_v], sem, add=True)

    def scatter_wait(rows, sem):
        pltpu.make_async_copy(rows, acc_sh.at[pl.ds(0, B)], sem).wait()

    R = (rows0, rows1, rows2)
    DV = (dst0, dst1, dst2)
    WV = (wv0, wv1, wv2)
    SG = (sg0, sg1, sg2)
    SD = (sd0, sd1, sd2)
    SW = (sw0, sw1, sw2)
    SS = (ss0, ss1, ss2)

    def step(b, i, has_prev_scatter, fire_next):
        j = (i + 1) % 3

        def waitj():
            scatter_wait(R[j], SS[j])

        if has_prev_scatter is True:
            waitj()
        elif has_prev_scatter is not False:
            pl.when(has_prev_scatter)(waitj)
        if fire_next:
            gather(b + 1, R[j], SG[j])
            dstload(b + 1, DV[j], SD[j])
            wload(b + 1, WV[j], SW[j])
        gather_wait(R[i], SG[i])
        w_wait(WV[i], SW[i])
        scale(R[i], WV[i])
        dst_wait(DV[i], SD[i])
        scatter(R[i], DV[i], SS[i])

    gather(0, rows0, sg0)
    dstload(0, dst0, sd0)
    wload(0, wv0, sw0)

    def triple_body(t, _):
        b0 = 3 * t
        step(b0 + 0, 0, t > 0, True)
        step(b0 + 1, 1, t > 0, True)
        step(b0 + 2, 2, True, True)
        return _

    lax.fori_loop(0, ntriple, triple_body, None)
    step(nbatch - 2, 0, True, True)
    step(nbatch - 1, 1, True, False)
    scatter_wait(R[0], SS[0])
    scatter_wait(R[1], SS[1])

    plsc.subcore_barrier()

    def wb_body(t, _):
        chunk = s + NS * t

        @pl.when(chunk < nchunk_rows)
        def _():
            r0 = chunk * B
            pltpu.sync_copy(acc_sh.at[pl.ds(r0, B)], rows0)
            pltpu.sync_copy(rows0, out_hbm.at[c, pl.ds(r0, B)])

        return _

    lax.fori_loop(0, nround, wb_body, None)


@functools.partial(jax.jit, static_argnames=("N", "E", "D"))
def _sc_scatter(x, src, dst, ew, N, E, D):
    B = 80
    NW = NC * NS
    NB = E // (NW * B)
    mesh = plsc.VectorSubcoreMesh(
        core_axis_name="c", subcore_axis_name="s",
        num_cores=NC, num_subcores=NS)
    f = pl.kernel(
        functools.partial(_sc_scatter_fn, N, E, D, B),
        out_type=jax.ShapeDtypeStruct((NC, N, D), jnp.float32),
        mesh=mesh,
        scratch_types=(
            [pltpu.VMEM((E // NW,), jnp.int32)]
            + [pltpu.VMEM((B,), jnp.int32) for _ in range(3)]
            + [pltpu.VMEM((B,), jnp.float32) for _ in range(3)]
            + [pltpu.VMEM((B, D), jnp.float32) for _ in range(3)]
            + [pltpu.VMEM_SHARED((N, D), jnp.float32)]
            + [pltpu.SemaphoreType.DMA for _ in range(12)]
        ),
    )
    return f(x, src, dst, ew)


def _tc_fn(p_ref, w_ref, b_ref, o_ref):
    a = p_ref[0] + p_ref[1]
    o_ref[...] = jnp.dot(a, w_ref[...],
                         preferred_element_type=jnp.float32) + b_ref[...]


@functools.partial(jax.jit, static_argnames=("bn",))
def _tc_finish(partials, weight, bias2d, bn):
    N, D = partials.shape[1], partials.shape[2]
    DO = weight.shape[1]
    grid = (N // bn,)
    return pl.pallas_call(
        _tc_fn,
        grid=grid,
        in_specs=[
            pl.BlockSpec((NC, bn, D), lambda i: (0, i, 0)),
            pl.BlockSpec((D, DO), lambda i: (0, 0)),
            pl.BlockSpec((1, DO), lambda i: (0, 0)),
        ],
        out_specs=pl.BlockSpec((bn, DO), lambda i: (i, 0)),
        out_shape=jax.ShapeDtypeStruct((N, DO), jnp.float32),
    )(partials, weight, bias2d)


def kernel(x, edge_index, edge_weight, weight, bias):
    N, D = x.shape
    E = edge_index.shape[1]
    ew = edge_weight.reshape(-1)
    src = edge_index[0]
    dst = edge_index[1]
    partials = _sc_scatter(x, src, dst, ew, N=N, E=E, D=D)
    return _tc_finish(partials, weight, bias.reshape(1, -1), bn=2000)
